# baseline (speedup 1.0000x reference)
import functools

import jax
import jax.numpy as jnp
from jax import lax
from jax.experimental import pallas as pl
from jax.experimental.pallas import tpu as pltpu
from jax.experimental.pallas import tpu_sc as plsc

D = 128
LANES = 16
NC = 2
NS = 16
CH = 128
SUP = 8
NJ = D // LANES


def _ceil_to(x, m):
    return (x + m - 1) // m * m


def _tc_pre_body(xq, xt, m, wq, wt, wm, y0, y1, y2, y3):
    f32 = jnp.float32
    y0[...] = jnp.dot(xq[...], wq[...], preferred_element_type=f32)
    y1[...] = jnp.dot(xt[...], wt[...], preferred_element_type=f32)
    y2[...] = lax.dot_general(xt[...], wm[...], (((1,), (1,)), ((), ())),
                              preferred_element_type=f32)
    y3[...] = jnp.dot(xq[...] * m[...], wm[...], preferred_element_type=f32)


def _tc_pre(Xq, Xt, maskf, Wq, Wt, Wm, n):
    bn = 1000
    grid = (n // bn,)
    row_spec = pl.BlockSpec((bn, D), lambda i: (i, 0))
    w_spec = pl.BlockSpec((D, D), lambda i: (0, 0))
    m_spec = pl.BlockSpec((bn, 1), lambda i: (i, 0))
    out = jax.ShapeDtypeStruct((n, D), jnp.float32)
    return pl.pallas_call(
        _tc_pre_body,
        grid=grid,
        in_specs=[row_spec, row_spec, m_spec, w_spec, w_spec, w_spec],
        out_specs=[row_spec, row_spec, row_spec, row_spec],
        out_shape=[out, out, out, out],
    )(Xq, Xt, maskf, Wq, Wt, Wm)


def _make_sc(n, npad, nsi, nsx):
    mesh = plsc.VectorSubcoreMesh(
        core_axis_name="c", subcore_axis_name="s", num_cores=NC,
        num_subcores=NS)
    rps = npad // NS
    nz = rps // CH

    @functools.partial(
        pl.kernel,
        out_type=jax.ShapeDtypeStruct((2, npad, D), jnp.float32),
        mesh=mesh,
        scratch_types=[
            pltpu.VMEM_SHARED((npad, D), jnp.float32),
            pltpu.VMEM((2, SUP, CH), jnp.int32),
            pltpu.VMEM((2, SUP, CH), jnp.int32),
            pltpu.VMEM((2, SUP, CH), jnp.float32),
            pltpu.VMEM((CH, D), jnp.float32),
            pltpu.VMEM((CH, D), jnp.float32),
            pltpu.SemaphoreType.DMA,
            pltpu.SemaphoreType.DMA,
            pltpu.SemaphoreType.DMA,
            pltpu.SemaphoreType.DMA,
            pltpu.SemaphoreType.DMA,
        ],
    )
    def sc_kernel(y_qi, y_ti, y_tc, y_qc,
                  eq_src, eq_dst, eq_nrm,
                  et_src, et_dst, et_nrm,
                  xg_q, xs_q, xg_t, xs_t,
                  out,
                  acc, gb, sb, nb, rows0, rows1,
                  semg0, semg1, sems0, sems1, semi):
        c = lax.axis_index("c")
        s = lax.axis_index("s")
        rows = (rows0, rows1)
        semg = (semg0, semg1)
        semsc = (sems0, sems1)
        zv = jnp.zeros((LANES,), jnp.float32)

        def idx_load(srcs, half, rb):
            return [pltpu.async_copy(r.at[pl.ds(rb, SUP)], b.at[half], semi)
                    for r, b in srcs]

        def process_super(y_ref, half, with_norm):
            dg = [None, None]
            dsc = [None, None]
            dg[0] = pltpu.async_copy(
                y_ref.at[gb.at[half, 0]], rows[0], semg[0])
            for k in range(SUP):
                p = k & 1
                if k + 1 < SUP:
                    if k >= 1:
                        dsc[1 - p].wait()
                    dg[1 - p] = pltpu.async_copy(
                        y_ref.at[gb.at[half, k + 1]], rows[1 - p],
                        semg[1 - p])
                dg[p].wait()
                if with_norm:
                    buf = rows[p]

                    def scale(e16, c2):
                        nv = nb[half, k, pl.ds(e16 * LANES, LANES)]
                        for l in range(LANES):
                            nvl = jnp.full((LANES,), nv[l])
                            e = e16 * LANES + l
                            for j in range(NJ):
                                sl = pl.ds(j * LANES, LANES)
                                buf[e, sl] = buf[e, sl] * nvl
                        return c2

                    lax.fori_loop(0, CH // LANES, scale, 0)
                dsc[p] = pltpu.async_copy(
                    rows[p], acc.at[sb.at[half, k]], semsc[p], add=True)
            dsc[0].wait()
            dsc[1].wait()

        def run_edges(y_ref, srcs, nsup, with_norm):
            base = s * nsup * SUP

            for d in idx_load(srcs, 0, base):
                d.wait()

            def body(i2, cc):
                i = 2 * i2
                dpre = idx_load(srcs, 1, base + (i + 1) * SUP)
                process_super(y_ref, 0, with_norm)
                for d in dpre:
                    d.wait()
                dpre2 = idx_load(srcs, 0, base + (i + 2) * SUP)
                process_super(y_ref, 1, with_norm)
                for d in dpre2:
                    d.wait()
                return cc

            lax.fori_loop(0, nsup // 2, body, 0)

        def run_graph(g, y_i, i_src, i_dst, i_nrm, y_c, c_g, c_s):
            def zr(e, cc):
                for j in range(NJ):
                    rows1[e, pl.ds(j * LANES, LANES)] = zv
                return cc

            lax.fori_loop(0, CH, zr, 0)
            for k in range(nz):
                pltpu.sync_copy(rows1, acc.at[pl.ds(s * rps + k * CH, CH)])
            plsc.subcore_barrier()

            run_edges(y_i, [(i_src, gb), (i_dst, sb), (i_nrm, nb)], nsi,
                      True)
            run_edges(y_c, [(c_g, gb), (c_s, sb)], nsx, False)
            plsc.subcore_barrier()

            for k in range(nz):
                off = s * rps + k * CH
                pltpu.sync_copy(acc.at[pl.ds(off, CH)], rows0)

                def rel(e, cc):
                    for j in range(NJ):
                        sl = pl.ds(j * LANES, LANES)
                        rows0[e, sl] = jnp.maximum(rows0[e, sl], 0.0)
                    return cc

                lax.fori_loop(0, CH, rel, 0)
                pltpu.sync_copy(rows0, out.at[g, pl.ds(off, CH)])

        @pl.when(c == 0)
        def _():
            run_graph(0, y_qi, eq_src, eq_dst, eq_nrm, y_tc, xg_q, xs_q)

        @pl.when(c == 1)
        def _():
            run_graph(1, y_ti, et_src, et_dst, et_nrm, y_qc, xg_t, xs_t)

    return sc_kernel


def _pad2d(a, total, val):
    e = a.shape[0]
    if e != total:
        a = jnp.concatenate([a, jnp.full((total - e,), val, a.dtype)])
    return a.reshape(-1, CH)


def kernel(Xq, edge_indexq, Xt, edge_indext, norm_q, norm_t, u2v_li,
           node_mask, only_inter, Wq, Wt, Wm):
    n = Xq.shape[0]
    npad = _ceil_to(n, NS * CH)
    dump = n + 8

    maskf = node_mask.astype(jnp.float32)[:, None]
    y_qi, y_ti, y_tc, y_qc = _tc_pre(Xq, Xt, maskf, Wq, Wt, Wm, n)

    intra_scale = jnp.where(jnp.asarray(only_inter) != 0, 0.0, 1.0)

    unit = NS * SUP * CH
    eq = edge_indexq.shape[1]
    et = edge_indext.shape[1]
    ex = u2v_li.shape[1]

    def _even_pad(e):
        ep = _ceil_to(e, unit)
        if (ep // unit) % 2:
            ep += unit
        return ep

    epq = _even_pad(eq)
    ept = _even_pad(et)
    epx = _even_pad(ex)
    guard = SUP * CH

    eq_src = _pad2d(edge_indexq[0], epq + guard, 0)
    eq_dst = _pad2d(edge_indexq[1], epq + guard, dump)
    eq_nrm = _pad2d(norm_q * intra_scale, epq + guard, 0.0)
    et_src = _pad2d(edge_indext[0], ept + guard, 0)
    et_dst = _pad2d(edge_indext[1], ept + guard, dump)
    et_nrm = _pad2d(norm_t * intra_scale, ept + guard, 0.0)
    u = u2v_li[0]
    v = u2v_li[1]
    xg_q = _pad2d(v, epx + guard, 0)
    xs_q = _pad2d(u, epx + guard, dump)
    xg_t = _pad2d(u, epx + guard, 0)
    xs_t = _pad2d(v, epx + guard, dump)

    sc = _make_sc(n, npad, epq // unit, epx // unit)
    assert ept == epq
    O = sc(y_qi, y_ti, y_tc, y_qc,
           eq_src, eq_dst, eq_nrm,
           et_src, et_dst, et_nrm,
           xg_q, xs_q, xg_t, xs_t)
    return (O[0, :n], O[1, :n])


# speedup vs baseline: 1.2761x; 1.2761x over previous
import functools

import jax
import jax.numpy as jnp
from jax import lax
from jax.experimental import pallas as pl
from jax.experimental.pallas import tpu as pltpu
from jax.experimental.pallas import tpu_sc as plsc

D = 128
LANES = 16
NC = 2
NS = 16
CH = 128
SUP = 8
NJ = D // LANES


def _ceil_to(x, m):
    return (x + m - 1) // m * m


def _tc_pre_body(xq, xt, m, wq, wt, wm, y0, y1, y2, y3):
    f32 = jnp.float32
    y0[...] = jnp.dot(xq[...], wq[...], preferred_element_type=f32)
    y1[...] = jnp.dot(xt[...], wt[...], preferred_element_type=f32)
    y2[...] = lax.dot_general(xt[...], wm[...], (((1,), (1,)), ((), ())),
                              preferred_element_type=f32)
    y3[...] = jnp.dot(xq[...] * m[...], wm[...], preferred_element_type=f32)


def _tc_pre(Xq, Xt, maskf, Wq, Wt, Wm, n):
    bn = 1000
    grid = (n // bn,)
    row_spec = pl.BlockSpec((bn, D), lambda i: (i, 0))
    w_spec = pl.BlockSpec((D, D), lambda i: (0, 0))
    m_spec = pl.BlockSpec((bn, 1), lambda i: (i, 0))
    out = jax.ShapeDtypeStruct((n, D), jnp.float32)
    return pl.pallas_call(
        _tc_pre_body,
        grid=grid,
        in_specs=[row_spec, row_spec, m_spec, w_spec, w_spec, w_spec],
        out_specs=[row_spec, row_spec, row_spec, row_spec],
        out_shape=[out, out, out, out],
    )(Xq, Xt, maskf, Wq, Wt, Wm)


def _make_sc(n, npad, nsi, nsx):
    mesh = plsc.VectorSubcoreMesh(
        core_axis_name="c", subcore_axis_name="s", num_cores=NC,
        num_subcores=NS)
    rps = npad // NS
    nz = rps // CH

    @functools.partial(
        pl.kernel,
        out_type=jax.ShapeDtypeStruct((2, npad, D), jnp.float32),
        mesh=mesh,
        scratch_types=[
            pltpu.VMEM_SHARED((npad, D), jnp.float32),
            pltpu.VMEM((2, SUP, CH), jnp.int32),
            pltpu.VMEM((2, SUP, CH), jnp.int32),
            pltpu.VMEM((2, SUP, CH), jnp.float32),
            pltpu.VMEM((CH, D), jnp.float32),
            pltpu.VMEM((CH, D), jnp.float32),
            pltpu.SemaphoreType.DMA,
            pltpu.SemaphoreType.DMA,
            pltpu.SemaphoreType.DMA,
            pltpu.SemaphoreType.DMA,
            pltpu.SemaphoreType.DMA,
        ],
    )
    def sc_kernel(y_qi, y_ti, y_tc, y_qc,
                  eq_src, eq_dst, eq_nrm,
                  et_src, et_dst, et_nrm,
                  xg_q, xs_q, xg_t, xs_t,
                  out,
                  acc, gb, sb, nb, rows0, rows1,
                  semg0, semg1, sems0, sems1, semi):
        c = lax.axis_index("c")
        s = lax.axis_index("s")
        rows = (rows0, rows1)
        semg = (semg0, semg1)
        semsc = (sems0, sems1)
        zv = jnp.zeros((LANES,), jnp.float32)

        def idx_load(srcs, half, rb):
            return [pltpu.async_copy(r.at[pl.ds(rb, SUP)], b.at[half], semi)
                    for r, b in srcs]

        def run_edges(y_ref, srcs, nsup, with_norm):
            base = s * nsup * SUP

            for d in idx_load(srcs, 0, base):
                d.wait()

            def body(i, cc):
                half = i & 1
                dpre = idx_load(srcs, 1 - half, base + (i + 1) * SUP)
                dg = [None, None]
                dg[0] = pltpu.async_copy(
                    y_ref.at[gb.at[half, 0]], rows[0], semg[0])
                for k in range(SUP):
                    p = k & 1
                    if k + 1 < SUP:
                        dg[1 - p] = pltpu.async_copy(
                            y_ref.at[gb.at[half, k + 1]], rows[1 - p],
                            semg[1 - p])
                    dg[p].wait()
                    if with_norm:
                        buf = rows[p]

                        def scale(e16, c2):
                            nv = nb[half, k, pl.ds(e16 * LANES, LANES)]
                            for l in range(LANES):
                                nvl = jnp.full((LANES,), nv[l])
                                e = e16 * LANES + l
                                for j in range(NJ):
                                    sl = pl.ds(j * LANES, LANES)
                                    buf[e, sl] = buf[e, sl] * nvl
                            return c2

                        lax.fori_loop(0, CH // LANES, scale, 0)
                    pltpu.sync_copy(rows[p], acc.at[sb.at[half, k]],
                                    add=True)
                for d in dpre:
                    d.wait()
                return cc

            lax.fori_loop(0, nsup, body, 0)

        def run_graph(g, y_i, i_src, i_dst, i_nrm, y_c, c_g, c_s):
            def zr(e, cc):
                for j in range(NJ):
                    rows1[e, pl.ds(j * LANES, LANES)] = zv
                return cc

            lax.fori_loop(0, CH, zr, 0)
            for k in range(nz):
                pltpu.sync_copy(rows1, acc.at[pl.ds(s * rps + k * CH, CH)])
            plsc.subcore_barrier()

            run_edges(y_i, [(i_src, gb), (i_dst, sb), (i_nrm, nb)], nsi,
                      True)
            run_edges(y_c, [(c_g, gb), (c_s, sb)], nsx, False)
            plsc.subcore_barrier()

            for k in range(nz):
                off = s * rps + k * CH
                pltpu.sync_copy(acc.at[pl.ds(off, CH)], rows0)

                def rel(e, cc):
                    for j in range(NJ):
                        sl = pl.ds(j * LANES, LANES)
                        rows0[e, sl] = jnp.maximum(rows0[e, sl], 0.0)
                    return cc

                lax.fori_loop(0, CH, rel, 0)
                pltpu.sync_copy(rows0, out.at[g, pl.ds(off, CH)])

        @pl.when(c == 0)
        def _():
            run_graph(0, y_qi, eq_src, eq_dst, eq_nrm, y_tc, xg_q, xs_q)

        @pl.when(c == 1)
        def _():
            run_graph(1, y_ti, et_src, et_dst, et_nrm, y_qc, xg_t, xs_t)

    return sc_kernel


def _pad2d(a, total, val):
    e = a.shape[0]
    if e != total:
        a = jnp.concatenate([a, jnp.full((total - e,), val, a.dtype)])
    return a.reshape(-1, CH)


def kernel(Xq, edge_indexq, Xt, edge_indext, norm_q, norm_t, u2v_li,
           node_mask, only_inter, Wq, Wt, Wm):
    n = Xq.shape[0]
    npad = _ceil_to(n, NS * CH)
    dump = n + 8

    maskf = node_mask.astype(jnp.float32)[:, None]
    y_qi, y_ti, y_tc, y_qc = _tc_pre(Xq, Xt, maskf, Wq, Wt, Wm, n)

    intra_scale = jnp.where(jnp.asarray(only_inter) != 0, 0.0, 1.0)

    unit = NS * SUP * CH
    eq = edge_indexq.shape[1]
    et = edge_indext.shape[1]
    ex = u2v_li.shape[1]

    epq = _ceil_to(eq, unit)
    ept = _ceil_to(et, unit)
    epx = _ceil_to(ex, unit)
    guard = SUP * CH

    eq_src = _pad2d(edge_indexq[0], epq + guard, 0)
    eq_dst = _pad2d(edge_indexq[1], epq + guard, dump)
    eq_nrm = _pad2d(norm_q * intra_scale, epq + guard, 0.0)
    et_src = _pad2d(edge_indext[0], ept + guard, 0)
    et_dst = _pad2d(edge_indext[1], ept + guard, dump)
    et_nrm = _pad2d(norm_t * intra_scale, ept + guard, 0.0)
    u = u2v_li[0]
    v = u2v_li[1]
    xg_q = _pad2d(v, epx + guard, 0)
    xs_q = _pad2d(u, epx + guard, dump)
    xg_t = _pad2d(u, epx + guard, 0)
    xs_t = _pad2d(v, epx + guard, dump)

    sc = _make_sc(n, npad, epq // unit, epx // unit)
    assert ept == epq
    O = sc(y_qi, y_ti, y_tc, y_qc,
           eq_src, eq_dst, eq_nrm,
           et_src, et_dst, et_nrm,
           xg_q, xs_q, xg_t, xs_t)
    return (O[0, :n], O[1, :n])


# speedup vs baseline: 2.3824x; 1.8669x over previous
import functools

import jax
import jax.numpy as jnp
from jax import lax
from jax.experimental import pallas as pl
from jax.experimental.pallas import tpu as pltpu
from jax.experimental.pallas import tpu_sc as plsc

D = 128
DH = D // 2
LANES = 16
NC = 2
NS = 16
CH = 128
SUP = 8
NJH = DH // LANES


def _ceil_to(x, m):
    return (x + m - 1) // m * m


def _tc_pre_body(xq, xt, m, wq, wt, wm, y0, y1, y2, y3):
    f32 = jnp.float32
    y0[...] = jnp.dot(xq[...], wq[...], preferred_element_type=f32)
    y1[...] = jnp.dot(xt[...], wt[...], preferred_element_type=f32)
    y2[...] = lax.dot_general(xt[...], wm[...], (((1,), (1,)), ((), ())),
                              preferred_element_type=f32)
    y3[...] = jnp.dot(xq[...] * m[...], wm[...], preferred_element_type=f32)


def _tc_pre(Xq, Xt, maskf, Wq, Wt, Wm, n):
    bn = 1000
    grid = (n // bn,)
    row_spec = pl.BlockSpec((bn, D), lambda i: (i, 0))
    w_spec = pl.BlockSpec((D, D), lambda i: (0, 0))
    m_spec = pl.BlockSpec((bn, 1), lambda i: (i, 0))
    out = jax.ShapeDtypeStruct((n, D), jnp.float32)
    return pl.pallas_call(
        _tc_pre_body,
        grid=grid,
        in_specs=[row_spec, row_spec, m_spec, w_spec, w_spec, w_spec],
        out_specs=[row_spec, row_spec, row_spec, row_spec],
        out_shape=[out, out, out, out],
    )(Xq, Xt, maskf, Wq, Wt, Wm)


def _make_sc(n, npad, nsi, nsx):
    mesh = plsc.VectorSubcoreMesh(
        core_axis_name="c", subcore_axis_name="s", num_cores=NC,
        num_subcores=NS)
    rps = npad // NS
    nz = rps // CH

    @functools.partial(
        pl.kernel,
        out_type=jax.ShapeDtypeStruct((2, 2, npad, DH), jnp.float32),
        mesh=mesh,
        scratch_types=[
            pltpu.VMEM_SHARED((npad, DH), jnp.float32),
            pltpu.VMEM_SHARED((npad, DH), jnp.float32),
            pltpu.VMEM((2, SUP, CH), jnp.int32),
            pltpu.VMEM((2, SUP, CH), jnp.int32),
            pltpu.VMEM((2, SUP, CH), jnp.float32),
            pltpu.VMEM((CH, DH), jnp.float32),
            pltpu.VMEM((CH, DH), jnp.float32),
            pltpu.SemaphoreType.DMA,
            pltpu.SemaphoreType.DMA,
            pltpu.SemaphoreType.DMA,
        ],
    )
    def sc_kernel(yqi0, yqi1, ytc0, ytc1, yti0, yti1, yqc0, yqc1,
                  eq_src, eq_dst, eq_nrm,
                  et_src, et_dst, et_nrm,
                  xg_q, xs_q, xg_t, xs_t,
                  out,
                  tbl, acc, gb, sb, nb, rows0, rows1,
                  semg0, semg1, semi):
        s = lax.axis_index("s")
        rows = (rows0, rows1)
        semg = (semg0, semg1)
        zv = jnp.zeros((LANES,), jnp.float32)

        def idx_load(srcs, half, rb):
            return [pltpu.async_copy(r.at[pl.ds(rb, SUP)], b.at[half], semi)
                    for r, b in srcs]

        def run_edges(srcs, nsup, with_norm):
            base = s * nsup * SUP

            for d in idx_load(srcs, 0, base):
                d.wait()

            def body(i, cc):
                half = i & 1
                dpre = idx_load(srcs, 1 - half, base + (i + 1) * SUP)
                dg = [None, None]
                dg[0] = pltpu.async_copy(
                    tbl.at[gb.at[half, 0]], rows[0], semg[0])
                for k in range(SUP):
                    p = k & 1
                    if k + 1 < SUP:
                        dg[1 - p] = pltpu.async_copy(
                            tbl.at[gb.at[half, k + 1]], rows[1 - p],
                            semg[1 - p])
                    dg[p].wait()
                    if with_norm:
                        buf = rows[p]

                        def scale(e16, c2):
                            nv = nb[half, k, pl.ds(e16 * LANES, LANES)]
                            for l in range(LANES):
                                nvl = jnp.full((LANES,), nv[l])
                                e = e16 * LANES + l
                                for j in range(NJH):
                                    sl = pl.ds(j * LANES, LANES)
                                    buf[e, sl] = buf[e, sl] * nvl
                            return c2

                        lax.fori_loop(0, CH // LANES, scale, 0)
                    pltpu.sync_copy(rows[p], acc.at[sb.at[half, k]],
                                    add=True)
                for d in dpre:
                    d.wait()
                return cc

            lax.fori_loop(0, nsup, body, 0)

        def stage(src_hbm):
            for k in range(nz):
                off = s * rps + k * CH
                pltpu.sync_copy(src_hbm.at[pl.ds(off, CH)], rows0)
                pltpu.sync_copy(rows0, tbl.at[pl.ds(off, CH)])

        def run_graph(g, y_i, i_src, i_dst, i_nrm, y_c, c_g, c_s):
            for h in range(2):
                def zr(e, cc):
                    for j in range(NJH):
                        rows1[e, pl.ds(j * LANES, LANES)] = zv
                    return cc

                lax.fori_loop(0, CH, zr, 0)
                for k in range(nz):
                    pltpu.sync_copy(
                        rows1, acc.at[pl.ds(s * rps + k * CH, CH)])
                stage(y_i[h])
                plsc.subcore_barrier()
                run_edges([(i_src, gb), (i_dst, sb), (i_nrm, nb)], nsi,
                          True)
                plsc.subcore_barrier()
                stage(y_c[h])
                plsc.subcore_barrier()
                run_edges([(c_g, gb), (c_s, sb)], nsx, False)
                plsc.subcore_barrier()
                for k in range(nz):
                    off = s * rps + k * CH
                    pltpu.sync_copy(acc.at[pl.ds(off, CH)], rows0)

                    def rel(e, cc):
                        for j in range(NJH):
                            sl = pl.ds(j * LANES, LANES)
                            rows0[e, sl] = jnp.maximum(rows0[e, sl], 0.0)
                        return cc

                    lax.fori_loop(0, CH, rel, 0)
                    pltpu.sync_copy(rows0, out.at[g, h, pl.ds(off, CH)])
                plsc.subcore_barrier()

        c = lax.axis_index("c")

        @pl.when(c == 0)
        def _():
            run_graph(0, (yqi0, yqi1), eq_src, eq_dst, eq_nrm,
                      (ytc0, ytc1), xg_q, xs_q)

        @pl.when(c == 1)
        def _():
            run_graph(1, (yti0, yti1), et_src, et_dst, et_nrm,
                      (yqc0, yqc1), xg_t, xs_t)

    return sc_kernel


def _pad2d(a, total, val):
    e = a.shape[0]
    if e != total:
        a = jnp.concatenate([a, jnp.full((total - e,), val, a.dtype)])
    return a.reshape(-1, CH)


def kernel(Xq, edge_indexq, Xt, edge_indext, norm_q, norm_t, u2v_li,
           node_mask, only_inter, Wq, Wt, Wm):
    n = Xq.shape[0]
    npad = _ceil_to(n, NS * CH)
    dump = n + 8

    maskf = node_mask.astype(jnp.float32)[:, None]
    y_qi, y_ti, y_tc, y_qc = _tc_pre(Xq, Xt, maskf, Wq, Wt, Wm, n)

    def _halves(y):
        yp = jnp.pad(y, ((0, npad - n), (0, 0)))
        return yp[:, :DH] + 0.0, yp[:, DH:] + 0.0

    yqi0, yqi1 = _halves(y_qi)
    yti0, yti1 = _halves(y_ti)
    ytc0, ytc1 = _halves(y_tc)
    yqc0, yqc1 = _halves(y_qc)

    intra_scale = jnp.where(jnp.asarray(only_inter) != 0, 0.0, 1.0)

    unit = NS * SUP * CH
    eq = edge_indexq.shape[1]
    et = edge_indext.shape[1]
    ex = u2v_li.shape[1]
    epq = _ceil_to(eq, unit)
    ept = _ceil_to(et, unit)
    epx = _ceil_to(ex, unit)
    guard = SUP * CH

    eq_src = _pad2d(edge_indexq[0], epq + guard, 0)
    eq_dst = _pad2d(edge_indexq[1], epq + guard, dump)
    eq_nrm = _pad2d(norm_q * intra_scale, epq + guard, 0.0)
    et_src = _pad2d(edge_indext[0], ept + guard, 0)
    et_dst = _pad2d(edge_indext[1], ept + guard, dump)
    et_nrm = _pad2d(norm_t * intra_scale, ept + guard, 0.0)
    u = u2v_li[0]
    v = u2v_li[1]
    xg_q = _pad2d(v, epx + guard, 0)
    xs_q = _pad2d(u, epx + guard, dump)
    xg_t = _pad2d(u, epx + guard, 0)
    xs_t = _pad2d(v, epx + guard, dump)

    sc = _make_sc(n, npad, epq // unit, epx // unit)
    assert ept == epq
    O = sc(yqi0, yqi1, ytc0, ytc1, yti0, yti1, yqc0, yqc1,
           eq_src, eq_dst, eq_nrm,
           et_src, et_dst, et_nrm,
           xg_q, xs_q, xg_t, xs_t)
    Oq = jnp.concatenate([O[0, 0], O[0, 1]], axis=1)
    Ot = jnp.concatenate([O[1, 0], O[1, 1]], axis=1)
    return (Oq[:n], Ot[:n])
